# Initial kernel scaffold; baseline (speedup 1.0000x reference)
import jax
import jax.numpy as jnp
from jax.experimental import pallas as pl


def kernel(points0, W, b, opac, idx):
    raise NotImplementedError("write your pallas kernel here")


# speedup vs baseline: 2.1726x; 2.1726x over previous
import functools

import jax
import jax.numpy as jnp
from jax.experimental import pallas as pl
from jax.experimental.pallas import tpu as pltpu


def _body(K, xy_ref, rx_ref, ry_ref, om_ref, w_ref, b_ref, o_ref, idx_ref,
          out_ref, xs, ys):
    i = pl.program_id(0)

    @pl.when(i == 0)
    def _init():
        xs[...] = xy_ref[0]
        ys[...] = xy_ref[1]

    rx = rx_ref[...]
    ry = ry_ref[...]
    om = om_ref[...]
    for k in range(K):
        t = i * K + k
        fi = idx_ref[t]
        w00 = w_ref[4 * fi]
        w01 = w_ref[4 * fi + 1]
        w10 = w_ref[4 * fi + 2]
        w11 = w_ref[4 * fi + 3]
        b0 = b_ref[2 * fi]
        b1 = b_ref[2 * fi + 1]
        op = o_ref[fi]
        x = xs[...]
        y = ys[...]
        xn = x * w00 + y * w01 + b0
        yn = x * w10 + y * w11 + b1
        xs[...] = xn
        ys[...] = yn
        slab = (jnp.dot(xn, rx, preferred_element_type=jnp.float32)
                + jnp.dot(yn, ry, preferred_element_type=jnp.float32)
                + op * om)
        out_ref[k] = slab


def kernel(points0, W, b, opac, idx):
    T = idx.shape[0]
    B = points0.shape[0]
    R = B // 128

    xy = points0.T.reshape(2, R, 128)

    cols = jnp.arange(384)[None, :]
    lanes = jnp.arange(128)[:, None]
    rx = jnp.where(cols == 3 * lanes, jnp.float32(0.25), jnp.float32(0.0))
    ry = jnp.where(cols == 3 * lanes + 1, jnp.float32(0.25), jnp.float32(0.0))
    om = (jnp.arange(384)[None, :] % 3 == 2).astype(jnp.float32)

    wf = W.reshape(-1).astype(jnp.float32)
    bf = b.reshape(-1).astype(jnp.float32)
    of = opac.astype(jnp.float32)
    idx32 = idx.astype(jnp.int32)

    K = 1
    for cand in (8, 10, 20, 25, 5, 4, 2):
        if T % cand == 0:
            K = cand
            break

    out = pl.pallas_call(
        functools.partial(_body, K),
        grid=(T // K,),
        in_specs=[
            pl.BlockSpec((2, R, 128), lambda i: (0, 0, 0)),
            pl.BlockSpec((128, 384), lambda i: (0, 0)),
            pl.BlockSpec((128, 384), lambda i: (0, 0)),
            pl.BlockSpec((1, 384), lambda i: (0, 0)),
            pl.BlockSpec(memory_space=pltpu.SMEM),
            pl.BlockSpec(memory_space=pltpu.SMEM),
            pl.BlockSpec(memory_space=pltpu.SMEM),
            pl.BlockSpec(memory_space=pltpu.SMEM),
        ],
        out_specs=pl.BlockSpec((K, R, 384), lambda i: (i, 0, 0)),
        out_shape=jax.ShapeDtypeStruct((T, R, 384), jnp.float32),
        scratch_shapes=[
            pltpu.VMEM((R, 128), jnp.float32),
            pltpu.VMEM((R, 128), jnp.float32),
        ],
    )(xy, rx, ry, om, wf, bf, of, idx32)
    return out.reshape(T * B, 3)


# speedup vs baseline: 52.5963x; 24.2087x over previous
import functools

import jax
import jax.numpy as jnp
from jax.experimental import pallas as pl
from jax.experimental.pallas import tpu as pltpu


def _body(K, R, xy_ref, e2_ref, w_ref, b_ref, o_ref, idx_ref, out_ref, xs, ys):
    i = pl.program_id(0)

    @pl.when(i == 0)
    def _init():
        xs[...] = xy_ref[0]
        ys[...] = xy_ref[1]

    e2 = e2_ref[...]
    for k in range(K):
        t = i * K + k
        fi = idx_ref[t]
        w00 = w_ref[4 * fi]
        w01 = w_ref[4 * fi + 1]
        w10 = w_ref[4 * fi + 2]
        w11 = w_ref[4 * fi + 3]
        b0 = b_ref[2 * fi]
        b1 = b_ref[2 * fi + 1]
        op = o_ref[fi]
        x = xs[...]
        y = ys[...]
        xn = x * w00 + y * w01 + b0
        yn = x * w10 + y * w11 + b1
        xs[...] = xn
        ys[...] = yn
        stacked = jnp.concatenate(
            [xn * 0.25, yn * 0.25, jnp.full((R, 128), op, jnp.float32)], axis=0)
        out_ref[k] = jnp.dot(e2, stacked, preferred_element_type=jnp.float32)


def kernel(points0, W, b, opac, idx):
    T = idx.shape[0]
    B = points0.shape[0]
    R = B // 128

    xy = points0.T.reshape(2, R, 128)

    rows = jnp.arange(4 * R)[:, None]
    srcs = jnp.arange(3 * R)[None, :]
    e2 = jnp.where(
        (rows % 4 == srcs // R) & (rows // 4 == srcs % R),
        jnp.float32(1.0), jnp.float32(0.0))

    wf = W.reshape(-1).astype(jnp.float32)
    bf = b.reshape(-1).astype(jnp.float32)
    of = opac.astype(jnp.float32)
    idx32 = idx.astype(jnp.int32)

    K = 1
    for cand in (8, 10, 20, 25, 5, 4, 2):
        if T % cand == 0:
            K = cand
            break

    out = pl.pallas_call(
        functools.partial(_body, K, R),
        grid=(T // K,),
        in_specs=[
            pl.BlockSpec((2, R, 128), lambda i: (0, 0, 0)),
            pl.BlockSpec((4 * R, 3 * R), lambda i: (0, 0)),
            pl.BlockSpec(memory_space=pltpu.SMEM),
            pl.BlockSpec(memory_space=pltpu.SMEM),
            pl.BlockSpec(memory_space=pltpu.SMEM),
            pl.BlockSpec(memory_space=pltpu.SMEM),
        ],
        out_specs=pl.BlockSpec((K, 4 * R, 128), lambda i: (i, 0, 0)),
        out_shape=jax.ShapeDtypeStruct((T, 4 * R, 128), jnp.float32),
        scratch_shapes=[
            pltpu.VMEM((R, 128), jnp.float32),
            pltpu.VMEM((R, 128), jnp.float32),
        ],
    )(xy, e2, wf, bf, of, idx32)
    v = out.reshape(T * B // 128, 4, 128)[:, :3, :]
    return v.transpose(0, 2, 1).reshape(T * B, 3)


# speedup vs baseline: 82.3736x; 1.5661x over previous
import functools

import jax
import jax.numpy as jnp
from jax.experimental import pallas as pl
from jax.experimental.pallas import tpu as pltpu


def _body(K, R, xy_ref, e2_ref, w_ref, b_ref, o_ref, idx_ref, out_ref, xs, ys):
    i = pl.program_id(0)

    @pl.when(i == 0)
    def _init():
        xs[...] = xy_ref[0]
        ys[...] = xy_ref[1]

    e2 = e2_ref[...]
    for k in range(K):
        t = i * K + k
        fi = idx_ref[t]
        w00 = w_ref[4 * fi]
        w01 = w_ref[4 * fi + 1]
        w10 = w_ref[4 * fi + 2]
        w11 = w_ref[4 * fi + 3]
        b0 = b_ref[2 * fi]
        b1 = b_ref[2 * fi + 1]
        op = o_ref[fi]
        x = xs[...]
        y = ys[...]
        xn = x * w00 + y * w01 + b0
        yn = x * w10 + y * w11 + b1
        xs[...] = xn
        ys[...] = yn
        stacked = jnp.concatenate(
            [xn * 0.25, yn * 0.25, jnp.full((R, 128), op, jnp.float32)], axis=0)
        out_ref[k] = jnp.dot(e2, stacked, preferred_element_type=jnp.float32)


def kernel(points0, W, b, opac, idx):
    T = idx.shape[0]
    B = points0.shape[0]
    R = B // 128

    xy = points0.T.reshape(2, R, 128)

    rows = jnp.arange(4 * R)[:, None]
    srcs = jnp.arange(3 * R)[None, :]
    e2 = jnp.where(
        (rows % 4 == srcs // R) & (rows // 4 == srcs % R),
        jnp.float32(1.0), jnp.float32(0.0))

    wf = W.reshape(-1).astype(jnp.float32)
    bf = b.reshape(-1).astype(jnp.float32)
    of = opac.astype(jnp.float32)
    idx32 = idx.astype(jnp.int32)

    K = 1
    for cand in (8, 10, 20, 25, 5, 4, 2):
        if T % cand == 0:
            K = cand
            break

    out = pl.pallas_call(
        functools.partial(_body, K, R),
        grid=(T // K,),
        in_specs=[
            pl.BlockSpec((2, R, 128), lambda i: (0, 0, 0)),
            pl.BlockSpec((4 * R, 3 * R), lambda i: (0, 0)),
            pl.BlockSpec(memory_space=pltpu.SMEM),
            pl.BlockSpec(memory_space=pltpu.SMEM),
            pl.BlockSpec(memory_space=pltpu.SMEM),
            pl.BlockSpec(memory_space=pltpu.SMEM),
        ],
        out_specs=pl.BlockSpec((K, 4 * R, 128), lambda i: (i, 0, 0)),
        out_shape=jax.ShapeDtypeStruct((T, 4 * R, 128), jnp.float32),
        scratch_shapes=[
            pltpu.VMEM((R, 128), jnp.float32),
            pltpu.VMEM((R, 128), jnp.float32),
        ],
    )(xy, e2, wf, bf, of, idx32)
    v = out.reshape(T * B // 128, 4, 128).transpose(0, 2, 1)
    return v.reshape(T * B, 4)[:, :3]


# speedup vs baseline: 141.1602x; 1.7137x over previous
import functools

import jax
import jax.numpy as jnp
from jax.experimental import pallas as pl
from jax.experimental.pallas import tpu as pltpu


def _body(K, R, xy_ref, e2_ref, w_ref, b_ref, o_ref, idx_ref, out_ref, xs, ys):
    i = pl.program_id(0)

    @pl.when(i == 0)
    def _init():
        xs[...] = xy_ref[0]
        ys[...] = xy_ref[1]

    e2 = e2_ref[...]
    for k in range(K):
        t = i * K + k
        fi = idx_ref[t]
        w00 = w_ref[4 * fi]
        w01 = w_ref[4 * fi + 1]
        w10 = w_ref[4 * fi + 2]
        w11 = w_ref[4 * fi + 3]
        b0 = b_ref[2 * fi]
        b1 = b_ref[2 * fi + 1]
        op = o_ref[fi]
        x = xs[...]
        y = ys[...]
        xn = x * w00 + y * w01 + b0
        yn = x * w10 + y * w11 + b1
        xs[...] = xn
        ys[...] = yn
        stacked = jnp.concatenate(
            [xn * 0.25, yn * 0.25, jnp.full((R, 128), op, jnp.float32)], axis=0)
        out_ref[k] = jnp.dot(e2, stacked, preferred_element_type=jnp.float32)


def kernel(points0, W, b, opac, idx):
    T = idx.shape[0]
    B = points0.shape[0]
    R = B // 128

    xy = points0.T.reshape(2, R, 128)

    rows = jnp.arange(4 * R)[:, None]
    srcs = jnp.arange(3 * R)[None, :]
    e2 = jnp.where(
        (rows % 4 == srcs // R) & (rows // 4 == srcs % R),
        jnp.float32(1.0), jnp.float32(0.0))

    wf = W.reshape(-1).astype(jnp.float32)
    bf = b.reshape(-1).astype(jnp.float32)
    of = opac.astype(jnp.float32)
    idx32 = idx.astype(jnp.int32)

    K = 1
    for cand in (20, 25, 10, 8, 5, 4, 2):
        if T % cand == 0:
            K = cand
            break

    out = pl.pallas_call(
        functools.partial(_body, K, R),
        grid=(T // K,),
        in_specs=[
            pl.BlockSpec((2, R, 128), lambda i: (0, 0, 0)),
            pl.BlockSpec((4 * R, 3 * R), lambda i: (0, 0)),
            pl.BlockSpec(memory_space=pltpu.SMEM),
            pl.BlockSpec(memory_space=pltpu.SMEM),
            pl.BlockSpec(memory_space=pltpu.SMEM),
            pl.BlockSpec(memory_space=pltpu.SMEM),
        ],
        out_specs=pl.BlockSpec((K, 4 * R, 128), lambda i: (i, 0, 0)),
        out_shape=jax.ShapeDtypeStruct((T, 4 * R, 128), jnp.float32),
        scratch_shapes=[
            pltpu.VMEM((R, 128), jnp.float32),
            pltpu.VMEM((R, 128), jnp.float32),
        ],
    )(xy, e2, wf, bf, of, idx32)
    v = out.reshape(T * B // 128, 4, 128).transpose(0, 2, 1)
    return v.reshape(T * B, 4)[:, :3]


# speedup vs baseline: 197.3089x; 1.3978x over previous
import functools

import jax
import jax.numpy as jnp
from jax.experimental import pallas as pl
from jax.experimental.pallas import tpu as pltpu


def _body(K, R, xy_ref, e2_ref, w_ref, b_ref, o_ref, idx_ref, out_ref, xs, ys):
    i = pl.program_id(0)

    @pl.when(i == 0)
    def _init():
        xs[...] = xy_ref[0]
        ys[...] = xy_ref[1]

    e2 = e2_ref[...]
    for k in range(K):
        t = i * K + k
        fi = idx_ref[t]
        w00 = w_ref[4 * fi]
        w01 = w_ref[4 * fi + 1]
        w10 = w_ref[4 * fi + 2]
        w11 = w_ref[4 * fi + 3]
        b0 = b_ref[2 * fi]
        b1 = b_ref[2 * fi + 1]
        op = o_ref[fi]
        x = xs[...]
        y = ys[...]
        xn = x * w00 + y * w01 + b0
        yn = x * w10 + y * w11 + b1
        xs[...] = xn
        ys[...] = yn
        stacked = jnp.concatenate(
            [xn * 0.25, yn * 0.25, jnp.full((R, 128), op, jnp.float32)], axis=0)
        out_ref[k] = jnp.dot(e2, stacked, preferred_element_type=jnp.float32)


def kernel(points0, W, b, opac, idx):
    T = idx.shape[0]
    B = points0.shape[0]
    R = B // 128

    xy = points0.T.reshape(2, R, 128)

    rows = jnp.arange(4 * R)[:, None]
    srcs = jnp.arange(3 * R)[None, :]
    e2 = jnp.where(
        (rows % 4 == srcs // R) & (rows // 4 == srcs % R),
        jnp.float32(1.0), jnp.float32(0.0))

    wf = W.reshape(-1).astype(jnp.float32)
    bf = b.reshape(-1).astype(jnp.float32)
    of = opac.astype(jnp.float32)
    idx32 = idx.astype(jnp.int32)

    K = 1
    for cand in (50, 40, 25, 20, 10, 8, 5, 4, 2):
        if T % cand == 0:
            K = cand
            break

    out = pl.pallas_call(
        functools.partial(_body, K, R),
        grid=(T // K,),
        in_specs=[
            pl.BlockSpec((2, R, 128), lambda i: (0, 0, 0)),
            pl.BlockSpec((4 * R, 3 * R), lambda i: (0, 0)),
            pl.BlockSpec(memory_space=pltpu.SMEM),
            pl.BlockSpec(memory_space=pltpu.SMEM),
            pl.BlockSpec(memory_space=pltpu.SMEM),
            pl.BlockSpec(memory_space=pltpu.SMEM),
        ],
        out_specs=pl.BlockSpec((K, 4 * R, 128), lambda i: (i, 0, 0)),
        out_shape=jax.ShapeDtypeStruct((T, 4 * R, 128), jnp.float32),
        scratch_shapes=[
            pltpu.VMEM((R, 128), jnp.float32),
            pltpu.VMEM((R, 128), jnp.float32),
        ],
    )(xy, e2, wf, bf, of, idx32)
    v = out.reshape(T * B // 128, 4, 128).transpose(0, 2, 1)
    return v.reshape(T * B, 4)[:, :3]


# speedup vs baseline: 212.8067x; 1.0785x over previous
import functools

import jax
import jax.numpy as jnp
from jax.experimental import pallas as pl
from jax.experimental.pallas import tpu as pltpu


def _body(K, R, xy_ref, e2_ref, w_ref, b_ref, o_ref, idx_ref, out_ref, xs, ys):
    i = pl.program_id(0)

    @pl.when(i == 0)
    def _init():
        xs[...] = xy_ref[0]
        ys[...] = xy_ref[1]

    e2 = e2_ref[...]
    for k in range(K):
        t = i * K + k
        fi = idx_ref[t]
        w00 = w_ref[4 * fi]
        w01 = w_ref[4 * fi + 1]
        w10 = w_ref[4 * fi + 2]
        w11 = w_ref[4 * fi + 3]
        b0 = b_ref[2 * fi]
        b1 = b_ref[2 * fi + 1]
        op = o_ref[fi]
        x = xs[...]
        y = ys[...]
        xn = x * w00 + y * w01 + b0
        yn = x * w10 + y * w11 + b1
        xs[...] = xn
        ys[...] = yn
        stacked = jnp.concatenate(
            [xn * 0.25, yn * 0.25, jnp.full((R, 128), op, jnp.float32)], axis=0)
        out_ref[k] = jnp.dot(e2, stacked, preferred_element_type=jnp.float32)


def kernel(points0, W, b, opac, idx):
    T = idx.shape[0]
    B = points0.shape[0]
    R = B // 128

    xy = points0.T.reshape(2, R, 128)

    rows = jnp.arange(4 * R)[:, None]
    srcs = jnp.arange(3 * R)[None, :]
    e2 = jnp.where(
        (rows % 4 == srcs // R) & (rows // 4 == srcs % R),
        jnp.float32(1.0), jnp.float32(0.0))

    wf = W.reshape(-1).astype(jnp.float32)
    bf = b.reshape(-1).astype(jnp.float32)
    of = opac.astype(jnp.float32)
    idx32 = idx.astype(jnp.int32)

    K = 1
    for cand in (125, 100, 50, 40, 25, 20, 10, 8, 5, 4, 2):
        if T % cand == 0:
            K = cand
            break

    out = pl.pallas_call(
        functools.partial(_body, K, R),
        grid=(T // K,),
        in_specs=[
            pl.BlockSpec((2, R, 128), lambda i: (0, 0, 0)),
            pl.BlockSpec((4 * R, 3 * R), lambda i: (0, 0)),
            pl.BlockSpec(memory_space=pltpu.SMEM),
            pl.BlockSpec(memory_space=pltpu.SMEM),
            pl.BlockSpec(memory_space=pltpu.SMEM),
            pl.BlockSpec(memory_space=pltpu.SMEM),
        ],
        out_specs=pl.BlockSpec((K, 4 * R, 128), lambda i: (i, 0, 0)),
        out_shape=jax.ShapeDtypeStruct((T, 4 * R, 128), jnp.float32),
        scratch_shapes=[
            pltpu.VMEM((R, 128), jnp.float32),
            pltpu.VMEM((R, 128), jnp.float32),
        ],
    )(xy, e2, wf, bf, of, idx32)
    v = out.reshape(T * B // 128, 4, 128).transpose(0, 2, 1)
    return v.reshape(T * B, 4)[:, :3]


# speedup vs baseline: 224.7366x; 1.0561x over previous
import functools

import jax
import jax.numpy as jnp
from jax.experimental import pallas as pl
from jax.experimental.pallas import tpu as pltpu


def _body(K, R, xy_ref, w_ref, b_ref, o_ref, idx_ref, out_ref, xs, ys):
    i = pl.program_id(0)

    @pl.when(i == 0)
    def _init():
        xs[...] = xy_ref[0]
        ys[...] = xy_ref[1]

    x = xs[...]
    y = ys[...]
    for k in range(K):
        t = i * K + k
        fi = idx_ref[t]
        w00 = w_ref[4 * fi]
        w01 = w_ref[4 * fi + 1]
        w10 = w_ref[4 * fi + 2]
        w11 = w_ref[4 * fi + 3]
        b0 = b_ref[2 * fi]
        b1 = b_ref[2 * fi + 1]
        op = o_ref[fi]
        xn = x * w00 + y * w01 + b0
        yn = x * w10 + y * w11 + b1
        out_ref[k, 0::4, :] = xn * 0.25
        out_ref[k, 1::4, :] = yn * 0.25
        out_ref[k, 2::4, :] = jnp.full((R, 128), op, jnp.float32)
        x = xn
        y = yn
    xs[...] = x
    ys[...] = y


def kernel(points0, W, b, opac, idx):
    T = idx.shape[0]
    B = points0.shape[0]
    R = B // 128

    xy = points0.T.reshape(2, R, 128)

    wf = W.reshape(-1).astype(jnp.float32)
    bf = b.reshape(-1).astype(jnp.float32)
    of = opac.astype(jnp.float32)
    idx32 = idx.astype(jnp.int32)

    K = 1
    for cand in (125, 100, 50, 40, 25, 20, 10, 8, 5, 4, 2):
        if T % cand == 0:
            K = cand
            break

    out = pl.pallas_call(
        functools.partial(_body, K, R),
        grid=(T // K,),
        in_specs=[
            pl.BlockSpec((2, R, 128), lambda i: (0, 0, 0)),
            pl.BlockSpec(memory_space=pltpu.SMEM),
            pl.BlockSpec(memory_space=pltpu.SMEM),
            pl.BlockSpec(memory_space=pltpu.SMEM),
            pl.BlockSpec(memory_space=pltpu.SMEM),
        ],
        out_specs=pl.BlockSpec((K, 4 * R, 128), lambda i: (i, 0, 0)),
        out_shape=jax.ShapeDtypeStruct((T, 4 * R, 128), jnp.float32),
        scratch_shapes=[
            pltpu.VMEM((R, 128), jnp.float32),
            pltpu.VMEM((R, 128), jnp.float32),
        ],
    )(xy, wf, bf, of, idx32)
    v = out.reshape(T * B // 128, 4, 128).transpose(0, 2, 1)
    return v.reshape(T * B, 4)[:, :3]
